# baseline (speedup 1.0000x reference)
import functools

import jax
import jax.numpy as jnp
from jax import lax
from jax.experimental import pallas as pl
from jax.experimental.pallas import tpu as pltpu
from jax.experimental.pallas import tpu_sc as plsc

BATCH = 1024
SEQ = 200
D = 64
NC, NS = 2, 16
NW = NC * NS
SPW = SEQ // 4
G = SPW // 2
NBUF = 4
LEAD = 2
STEPS = G + LEAD
WROWS = SEQ * D

_mesh = plsc.VectorSubcoreMesh(core_axis_name="c", subcore_axis_name="s")

_scratch = (
    [pltpu.VMEM((WROWS,), jnp.float32)]
    + [pltpu.VMEM((7, 8, 128), jnp.int32)]
    + [pltpu.VMEM((2, 8, 8, 128), jnp.float32) for _ in range(NBUF)]
    + [pltpu.SemaphoreType.DMA for _ in range(2 * NBUF)]
)


@functools.partial(
    pl.kernel,
    mesh=_mesh,
    compiler_params=pltpu.CompilerParams(needs_layout_passes=False),
    out_type=jax.ShapeDtypeStruct((SEQ, 8, 8, 8, 128), jnp.float32),
    scratch_types=_scratch,
)
def _emb_add(x_hbm, p_hbm, w_hbm, out_hbm, wt_v, idx_v, *rest):
    slabs = list(rest[:NBUF])
    sin = list(rest[NBUF:2 * NBUF])
    sout = list(rest[2 * NBUF:3 * NBUF])

    wid = lax.axis_index("s") * NC + lax.axis_index("c")
    bt = wid % 8
    s_base = (wid // 8) * SPW
    sh0 = s_base // 8
    pltpu.sync_copy(w_hbm, wt_v)
    pltpu.sync_copy(p_hbm.at[pl.ds(sh0, 7), bt], idx_v)

    def start_in(i, b):
        s = s_base + 2 * i
        pltpu.async_copy(x_hbm.at[pl.ds(s, 2), :, bt], slabs[b], sin[b])

    def wait_in(b):
        pltpu.make_async_copy(x_hbm.at[pl.ds(0, 2), :, 0], slabs[b], sin[b]).wait()

    def start_o(i, b):
        s = s_base + 2 * i
        pltpu.async_copy(slabs[b], out_hbm.at[pl.ds(s, 2), :, bt], sout[b])

    def wait_o(b):
        pltpu.make_async_copy(x_hbm.at[pl.ds(0, 2), :, 0], slabs[b], sout[b]).wait()

    def compute(i, b):
        for k in range(2):
            soff = s_base + 2 * i + k - sh0 * 8

            def lgroup(l, c):
                sl = pl.ds(l * 16, 16)
                rbase = idx_v[soff // 8, soff % 8, sl] * D

                @plsc.parallel_loop(0, D, unroll=16)
                def _(e):
                    v = plsc.load_gather(wt_v, [rbase + e])
                    plsc.addupdate(slabs[b].at[k, e // 8, e % 8, sl], v)

                return c

            lax.fori_loop(0, 8, lgroup, 0)

    def outer(i2, carry):
        for u in range(NBUF):
            i = i2 * NBUF + u

            @pl.when(i < G)
            def _():
                @pl.when(i >= NBUF)
                def _():
                    wait_o(u)
                start_in(i, u)

            ib = i - LEAD
            bb = (u - LEAD) % NBUF

            @pl.when(jnp.logical_and(ib >= 0, ib < G))
            def _():
                wait_in(bb)
                compute(ib, bb)
                start_o(ib, bb)

        return carry

    lax.fori_loop(0, (STEPS + NBUF - 1) // NBUF, outer, 0)

    for b in range(NBUF):
        wait_o(b)


def kernel(x, pos_seq, position_weight):
    xv = (x.transpose(1, 2, 0)
          .reshape(SEQ, 8, 8, 8, 128)
          .transpose(0, 1, 3, 2, 4))
    pv = (pos_seq.T
          .reshape(SEQ // 8, 8, 8, 128)
          .transpose(0, 2, 1, 3))
    wf = position_weight.reshape(WROWS)
    o5 = _emb_add(xv, pv, wf)
    return (o5.transpose(0, 1, 3, 2, 4)
            .reshape(SEQ, D, BATCH)
            .transpose(2, 0, 1))


# speedup vs baseline: 3.1118x; 3.1118x over previous
import functools

import jax
import jax.numpy as jnp
from jax import lax
from jax.experimental import pallas as pl
from jax.experimental.pallas import tpu as pltpu
from jax.experimental.pallas import tpu_sc as plsc

BATCH = 1024
SEQ = 200
D = 64
NC, NS = 2, 16
NW = NC * NS
SPW = SEQ // 4
G = SPW // 2
NBUF = 4
LEAD = 2
STEPS = G + LEAD
WROWS = SEQ * D

_mesh = plsc.VectorSubcoreMesh(core_axis_name="c", subcore_axis_name="s")

_scratch = (
    [pltpu.VMEM((WROWS,), jnp.float32)]
    + [pltpu.VMEM((7, 8, 128), jnp.int32)]
    + [pltpu.VMEM((2, 8, 8, 128), jnp.float32) for _ in range(NBUF)]
    + [pltpu.SemaphoreType.DMA for _ in range(2 * NBUF)]
)


@functools.partial(
    pl.kernel,
    mesh=_mesh,
    compiler_params=pltpu.CompilerParams(needs_layout_passes=False),
    out_type=jax.ShapeDtypeStruct((SEQ, 8, 8, 8, 128), jnp.float32),
    scratch_types=_scratch,
)
def _emb_add(x_hbm, p_hbm, w_hbm, out_hbm, wt_v, idx_v, *rest):
    slabs = list(rest[:NBUF])
    sin = list(rest[NBUF:2 * NBUF])
    sout = list(rest[2 * NBUF:3 * NBUF])

    wid = lax.axis_index("s") * NC + lax.axis_index("c")
    bt = wid % 8
    s_base = (wid // 8) * SPW
    sh0 = s_base // 8
    pltpu.sync_copy(w_hbm, wt_v)
    pltpu.sync_copy(p_hbm.at[pl.ds(sh0, 7), bt], idx_v)

    def start_in(i, b):
        s = s_base + 2 * i
        pltpu.async_copy(x_hbm.at[pl.ds(s, 2), :, bt], slabs[b], sin[b])

    def wait_in(b):
        pltpu.make_async_copy(x_hbm.at[pl.ds(0, 2), :, 0], slabs[b], sin[b]).wait()

    def start_o(i, b):
        s = s_base + 2 * i
        pltpu.async_copy(slabs[b], out_hbm.at[pl.ds(s, 2), :, bt], sout[b])

    def wait_o(b):
        pltpu.make_async_copy(x_hbm.at[pl.ds(0, 2), :, 0], slabs[b], sout[b]).wait()

    def compute(i, b):
        for k in range(2):
            soff = s_base + 2 * i + k - sh0 * 8

            def lgroup(l, c):
                sl = pl.ds(l * 16, 16)
                rvec = idx_v[soff // 8, soff % 8, sl]

                @plsc.parallel_loop(0, D, unroll=16)
                def _(e):
                    v = plsc.load_gather(wt_v, [rvec + e * SEQ])
                    plsc.addupdate(slabs[b].at[k, e // 8, e % 8, sl], v)

                return c

            lax.fori_loop(0, 8, lgroup, 0)

    def outer(i2, carry):
        for u in range(NBUF):
            i = i2 * NBUF + u

            @pl.when(i < G)
            def _():
                @pl.when(i >= NBUF)
                def _():
                    wait_o(u)
                start_in(i, u)

            ib = i - LEAD
            bb = (u - LEAD) % NBUF

            @pl.when(jnp.logical_and(ib >= 0, ib < G))
            def _():
                wait_in(bb)
                compute(ib, bb)
                start_o(ib, bb)

        return carry

    lax.fori_loop(0, (STEPS + NBUF - 1) // NBUF, outer, 0)

    for b in range(NBUF):
        wait_o(b)


def kernel(x, pos_seq, position_weight):
    xv = (x.transpose(1, 2, 0)
          .reshape(SEQ, 8, 8, 8, 128)
          .transpose(0, 1, 3, 2, 4))
    pv = (pos_seq.T
          .reshape(SEQ // 8, 8, 8, 128)
          .transpose(0, 2, 1, 3))
    wf = position_weight.T.reshape(WROWS)
    o5 = _emb_add(xv, pv, wf)
    return (o5.transpose(0, 1, 3, 2, 4)
            .reshape(SEQ, D, BATCH)
            .transpose(2, 0, 1))
